# speedup vs baseline: 1.4464x; 1.4464x over previous
import jax
import jax.numpy as jnp
from jax import lax
from jax.experimental import pallas as pl
from jax.experimental.pallas import tpu as pltpu
from jax.experimental.pallas import tpu_sc as plsc

VOCAB = 24
EMBED_DIM = 128
BATCH = 256
SEQ = 1024

NC = 2
NS = 16
NW = NC * NS

TOTAL = BATCH * SEQ
PER_W = TOTAL // NW
GROUP = 128
NGROUPS = PER_W // GROUP


def _emb_kernel(table_hbm, idx_hbm, out_hbm, idx_v, rows_v, sem):
    wid = lax.axis_index("s") * NC + lax.axis_index("c")
    pltpu.sync_copy(idx_hbm.at[wid], idx_v)
    base = wid * PER_W

    def body(g, _):
        pltpu.async_copy(table_hbm.at[idx_v.at[g]], rows_v, sem).wait()
        pltpu.sync_copy(rows_v, out_hbm.at[pl.ds(base + g * GROUP, GROUP)])
        return 0

    lax.fori_loop(0, NGROUPS, body, 0)


def kernel(x, table):
    idx = x.reshape(NW, NGROUPS, GROUP)
    mesh = plsc.VectorSubcoreMesh(core_axis_name="c", subcore_axis_name="s")
    out = pl.kernel(
        _emb_kernel,
        mesh=mesh,
        out_type=jax.ShapeDtypeStruct((TOTAL, EMBED_DIM), jnp.float32),
        scratch_types=[
            pltpu.VMEM((NGROUPS, GROUP), jnp.int32),
            pltpu.VMEM((GROUP, EMBED_DIM), jnp.float32),
            pltpu.SemaphoreType.DMA,
        ],
    )(table, idx)
    return out.reshape(BATCH, SEQ, EMBED_DIM)


# speedup vs baseline: 1.4539x; 1.0052x over previous
import jax
import jax.numpy as jnp
from jax import lax
from jax.experimental import pallas as pl
from jax.experimental.pallas import tpu as pltpu
from jax.experimental.pallas import tpu_sc as plsc

VOCAB = 24
EMBED_DIM = 128
BATCH = 256
SEQ = 1024

NC = 2
NS = 16
NW = NC * NS

TOTAL = BATCH * SEQ
PER_W = TOTAL // NW
GROUP = 128
NGROUPS = PER_W // GROUP
DEPTH = 6
WOUT = 2


def _emb_kernel(table_hbm, idx_hbm, out_hbm, idx_v, rows_v, gsem, osem):
    wid = lax.axis_index("s") * NC + lax.axis_index("c")
    pltpu.sync_copy(idx_hbm.at[wid], idx_v)
    base = wid * PER_W

    def fire_gather(g, b):
        pltpu.async_copy(table_hbm.at[idx_v.at[g]], rows_v.at[b], gsem)

    def wait_gather(b):
        pltpu.make_async_copy(table_hbm.at[idx_v.at[0]], rows_v.at[b], gsem).wait()

    def fire_out(g, b):
        pltpu.async_copy(rows_v.at[b], out_hbm.at[pl.ds(base + g * GROUP, GROUP)], osem)

    def wait_out():
        pltpu.make_async_copy(
            rows_v.at[0], out_hbm.at[pl.ds(base, GROUP)], osem
        ).wait()

    for g in range(DEPTH - WOUT):
        fire_gather(g, g)

    def body(g, _):
        b = lax.rem(g, DEPTH)
        wait_gather(b)
        fire_out(g, b)

        @pl.when(g >= WOUT)
        def _():
            wait_out()

        @pl.when(g + DEPTH - WOUT < NGROUPS)
        def _():
            fire_gather(g + DEPTH - WOUT, lax.rem(g + DEPTH - WOUT, DEPTH))

        return 0

    lax.fori_loop(0, NGROUPS, body, 0)

    for _ in range(WOUT):
        wait_out()


def kernel(x, table):
    idx = x.reshape(NW, NGROUPS, GROUP)
    mesh = plsc.VectorSubcoreMesh(core_axis_name="c", subcore_axis_name="s")
    out = pl.kernel(
        _emb_kernel,
        mesh=mesh,
        out_type=jax.ShapeDtypeStruct((TOTAL, EMBED_DIM), jnp.float32),
        scratch_types=[
            pltpu.VMEM((NGROUPS, GROUP), jnp.int32),
            pltpu.VMEM((DEPTH, GROUP, EMBED_DIM), jnp.float32),
            pltpu.SemaphoreType.DMA,
            pltpu.SemaphoreType.DMA,
        ],
    )(table, idx)
    return out.reshape(BATCH, SEQ, EMBED_DIM)


# speedup vs baseline: 4.9517x; 3.4058x over previous
import jax
import jax.numpy as jnp
from jax import lax
from jax.experimental import pallas as pl
from jax.experimental.pallas import tpu as pltpu
from jax.experimental.pallas import tpu_sc as plsc

VOCAB = 24
EMBED_DIM = 128
BATCH = 256
SEQ = 1024

NC = 2
NS = 16
NW = NC * NS

TOTAL = BATCH * SEQ
PER_W = TOTAL // NW
GROUP = 128
NGROUPS = PER_W // GROUP
DEPTH = 6
WOUT = 2


def _emb_kernel(table_hbm, idx_hbm, out_hbm, idx_v, rows_v, gsem, osem):
    wid = lax.axis_index("s") * NC + lax.axis_index("c")
    pltpu.sync_copy(idx_hbm.at[wid], idx_v)
    base = wid * PER_W
    my_table = table_hbm.at[wid]

    def fire_gather(g, b):
        pltpu.async_copy(my_table.at[idx_v.at[g]], rows_v.at[b], gsem)

    def wait_gather(b):
        pltpu.make_async_copy(my_table.at[idx_v.at[0]], rows_v.at[b], gsem).wait()

    def fire_out(g, b):
        pltpu.async_copy(rows_v.at[b], out_hbm.at[pl.ds(base + g * GROUP, GROUP)], osem)

    def wait_out():
        pltpu.make_async_copy(
            rows_v.at[0], out_hbm.at[pl.ds(base, GROUP)], osem
        ).wait()

    for g in range(DEPTH - WOUT):
        fire_gather(g, g)

    def body(g, _):
        b = lax.rem(g, DEPTH)
        wait_gather(b)
        fire_out(g, b)

        @pl.when(g >= WOUT)
        def _():
            wait_out()

        @pl.when(g + DEPTH - WOUT < NGROUPS)
        def _():
            fire_gather(g + DEPTH - WOUT, lax.rem(g + DEPTH - WOUT, DEPTH))

        return 0

    lax.fori_loop(0, NGROUPS, body, 0)

    for _ in range(WOUT):
        wait_out()


def kernel(x, table):
    idx = x.reshape(NW, NGROUPS, GROUP)
    table_rep = jnp.tile(table[None], (NW, 1, 1))
    mesh = plsc.VectorSubcoreMesh(core_axis_name="c", subcore_axis_name="s")
    out = pl.kernel(
        _emb_kernel,
        mesh=mesh,
        out_type=jax.ShapeDtypeStruct((TOTAL, EMBED_DIM), jnp.float32),
        scratch_types=[
            pltpu.VMEM((NGROUPS, GROUP), jnp.int32),
            pltpu.VMEM((DEPTH, GROUP, EMBED_DIM), jnp.float32),
            pltpu.SemaphoreType.DMA,
            pltpu.SemaphoreType.DMA,
        ],
    )(table_rep, idx)
    return out.reshape(BATCH, SEQ, EMBED_DIM)


# speedup vs baseline: 6.9390x; 1.4013x over previous
import jax
import jax.numpy as jnp
from jax import lax
from jax.experimental import pallas as pl
from jax.experimental.pallas import tpu as pltpu
from jax.experimental.pallas import tpu_sc as plsc

VOCAB = 24
EMBED_DIM = 128
BATCH = 256
SEQ = 1024

NC = 2
NS = 16
NW = NC * NS

TOTAL = BATCH * SEQ
PER_W = TOTAL // NW
GROUP = 128
NGROUPS = PER_W // GROUP
DEPTH = 6
WOUT = 2
REPS = 4


def _emb_kernel(table_hbm, idx_hbm, out_hbm, idx_v, rows_v, gsem, osem):
    wid = lax.axis_index("s") * NC + lax.axis_index("c")
    pltpu.sync_copy(idx_hbm.at[wid], idx_v)
    base = wid * PER_W
    def fire_gather(g, b):
        rep = wid * REPS + lax.rem(g, REPS)
        pltpu.async_copy(table_hbm.at[rep].at[idx_v.at[g]], rows_v.at[b], gsem)

    def wait_gather(b):
        pltpu.make_async_copy(
            table_hbm.at[0].at[idx_v.at[0]], rows_v.at[b], gsem
        ).wait()

    def fire_out(g, b):
        pltpu.async_copy(rows_v.at[b], out_hbm.at[pl.ds(base + g * GROUP, GROUP)], osem)

    def wait_out():
        pltpu.make_async_copy(
            rows_v.at[0], out_hbm.at[pl.ds(base, GROUP)], osem
        ).wait()

    for g in range(DEPTH - WOUT):
        fire_gather(g, g)

    def body(g, _):
        b = lax.rem(g, DEPTH)
        wait_gather(b)
        fire_out(g, b)

        @pl.when(g >= WOUT)
        def _():
            wait_out()

        @pl.when(g + DEPTH - WOUT < NGROUPS)
        def _():
            fire_gather(g + DEPTH - WOUT, lax.rem(g + DEPTH - WOUT, DEPTH))

        return 0

    lax.fori_loop(0, NGROUPS, body, 0)

    for _ in range(WOUT):
        wait_out()


def kernel(x, table):
    idx = x.reshape(NW, NGROUPS, GROUP)
    table_rep = jnp.tile(table[None], (NW * REPS, 1, 1))
    mesh = plsc.VectorSubcoreMesh(core_axis_name="c", subcore_axis_name="s")
    out = pl.kernel(
        _emb_kernel,
        mesh=mesh,
        out_type=jax.ShapeDtypeStruct((TOTAL, EMBED_DIM), jnp.float32),
        scratch_types=[
            pltpu.VMEM((NGROUPS, GROUP), jnp.int32),
            pltpu.VMEM((DEPTH, GROUP, EMBED_DIM), jnp.float32),
            pltpu.SemaphoreType.DMA,
            pltpu.SemaphoreType.DMA,
        ],
    )(table_rep, idx)
    return out.reshape(BATCH, SEQ, EMBED_DIM)


# speedup vs baseline: 7.2457x; 1.0442x over previous
import jax
import jax.numpy as jnp
from jax import lax
from jax.experimental import pallas as pl
from jax.experimental.pallas import tpu as pltpu
from jax.experimental.pallas import tpu_sc as plsc

VOCAB = 24
EMBED_DIM = 128
BATCH = 256
SEQ = 1024

NC = 2
NS = 16
NW = NC * NS

TOTAL = BATCH * SEQ
PER_W = TOTAL // NW
GROUP = 128
NGROUPS = PER_W // GROUP
DEPTH = 6
WOUT = 2
REPS = 8


def _emb_kernel(table_hbm, idx_hbm, out_hbm, idx_v, rows_v, gsem, osem):
    wid = lax.axis_index("s") * NC + lax.axis_index("c")
    pltpu.sync_copy(idx_hbm.at[wid], idx_v)
    base = wid * PER_W
    def fire_gather(g, b):
        rep = wid * REPS + lax.rem(g, REPS)
        pltpu.async_copy(table_hbm.at[rep].at[idx_v.at[g]], rows_v.at[b], gsem)

    def wait_gather(b):
        pltpu.make_async_copy(
            table_hbm.at[0].at[idx_v.at[0]], rows_v.at[b], gsem
        ).wait()

    def fire_out(g, b):
        pltpu.async_copy(rows_v.at[b], out_hbm.at[pl.ds(base + g * GROUP, GROUP)], osem)

    def wait_out():
        pltpu.make_async_copy(
            rows_v.at[0], out_hbm.at[pl.ds(base, GROUP)], osem
        ).wait()

    for g in range(DEPTH - WOUT):
        fire_gather(g, g)

    def body(g, _):
        b = lax.rem(g, DEPTH)
        wait_gather(b)
        fire_out(g, b)

        @pl.when(g >= WOUT)
        def _():
            wait_out()

        @pl.when(g + DEPTH - WOUT < NGROUPS)
        def _():
            fire_gather(g + DEPTH - WOUT, lax.rem(g + DEPTH - WOUT, DEPTH))

        return 0

    lax.fori_loop(0, NGROUPS, body, 0)

    for _ in range(WOUT):
        wait_out()


def kernel(x, table):
    idx = x.reshape(NW, NGROUPS, GROUP)
    table_rep = jnp.tile(table[None], (NW * REPS, 1, 1))
    mesh = plsc.VectorSubcoreMesh(core_axis_name="c", subcore_axis_name="s")
    out = pl.kernel(
        _emb_kernel,
        mesh=mesh,
        out_type=jax.ShapeDtypeStruct((TOTAL, EMBED_DIM), jnp.float32),
        scratch_types=[
            pltpu.VMEM((NGROUPS, GROUP), jnp.int32),
            pltpu.VMEM((DEPTH, GROUP, EMBED_DIM), jnp.float32),
            pltpu.SemaphoreType.DMA,
            pltpu.SemaphoreType.DMA,
        ],
    )(table_rep, idx)
    return out.reshape(BATCH, SEQ, EMBED_DIM)
